# Initial kernel scaffold; baseline (speedup 1.0000x reference)
import jax
import jax.numpy as jnp
from jax.experimental import pallas as pl


def kernel(x, edge_index, seq_batch_node_id, edge_attr, edge_w0, edge_b0, mlp_w0, mlp_b0, bn_g0, bn_b0, edge_w1, edge_b1, mlp_w1, mlp_b1, bn_g1, bn_b1):
    raise NotImplementedError("write your pallas kernel here")


# speedup vs baseline: 3.3206x; 3.3206x over previous
import functools

import jax
import jax.numpy as jnp
from jax import lax
from jax.experimental import pallas as pl
from jax.experimental.pallas import tpu as pltpu
from jax.experimental.pallas import tpu_sc as plsc

N = 10000
E = 640000
D = 128
G = 16

NC = 2
NS = 16
NW = NC * NS
EPW = E // NW
C = 80
CH = EPW // C
ZROWS = N // C
RPT = N // NS


def _sc_body(x_hbm, e_hbm, src_hbm, dst_hbm, out_hbm,
             sidx, didx, rows, erows, agg, sem):
    c = lax.axis_index("c")
    s = lax.axis_index("s")
    wbase = (c * NS + s) * EPW

    def _zrow(r, _):
        zv = jnp.zeros((16,), jnp.float32)
        for j in range(8):
            rows[r, pl.ds(j * 16, 16)] = zv
        return 0
    lax.fori_loop(0, C, _zrow, 0)
    for jj in range(ZROWS // NS + 1):
        j = s + jj * NS
        @pl.when(j < ZROWS)
        def _():
            pltpu.sync_copy(rows, agg.at[pl.ds(j * C, C)])
    plsc.subcore_barrier()

    def _chunk(i, _):
        base = wbase + i * C
        pltpu.sync_copy(src_hbm.at[pl.ds(base, C)], sidx)
        pltpu.sync_copy(dst_hbm.at[pl.ds(base, C)], didx)
        pltpu.async_copy(x_hbm.at[sidx], rows, sem).wait()
        pltpu.sync_copy(e_hbm.at[pl.ds(base, C)], erows)

        def _row(r, _):
            for j in range(8):
                v = rows[r, pl.ds(j * 16, 16)] + erows[r, pl.ds(j * 16, 16)]
                rows[r, pl.ds(j * 16, 16)] = jnp.maximum(v, 0.0)
            return 0
        lax.fori_loop(0, C, _row, 0)

        pltpu.sync_copy(rows, agg.at[didx], add=True)
        return 0
    lax.fori_loop(0, CH, _chunk, 0)

    plsc.subcore_barrier()
    for jj in range(ZROWS // NS + 1):
        j = s + jj * NS
        @pl.when(j < ZROWS)
        def _():
            pltpu.sync_copy(agg.at[pl.ds(j * C, C)],
                            out_hbm.at[c, pl.ds(j * C, C)])


_sc_message_pass = functools.partial(
    pl.kernel,
    out_type=jax.ShapeDtypeStruct((NC, N, D), jnp.float32),
    mesh=plsc.VectorSubcoreMesh(
        core_axis_name="c", subcore_axis_name="s",
        num_cores=NC, num_subcores=NS),
    scratch_types=[
        pltpu.VMEM((C,), jnp.int32),
        pltpu.VMEM((C,), jnp.int32),
        pltpu.VMEM((C, D), jnp.float32),
        pltpu.VMEM((C, D), jnp.float32),
        pltpu.VMEM_SHARED((N, D), jnp.float32),
        pltpu.SemaphoreType.DMA,
    ],
)(_sc_body)


EB = 2000


def _edge_body(ea, w0, b0, w1, b1, e0, e1):
    a = ea[...]
    e0[...] = jnp.dot(a, w0[...], preferred_element_type=jnp.float32) + b0[...]
    e1[...] = jnp.dot(a, w1[...], preferred_element_type=jnp.float32) + b1[...]


def _edge_transform(edge_attr, ew0, eb0, ew1, eb1):
    de = edge_attr.shape[1]
    grid = E // EB
    return pl.pallas_call(
        _edge_body,
        grid=(grid,),
        in_specs=[
            pl.BlockSpec((EB, de), lambda i: (i, 0)),
            pl.BlockSpec((de, D), lambda i: (0, 0)),
            pl.BlockSpec((1, D), lambda i: (0, 0)),
            pl.BlockSpec((de, D), lambda i: (0, 0)),
            pl.BlockSpec((1, D), lambda i: (0, 0)),
        ],
        out_specs=[
            pl.BlockSpec((EB, D), lambda i: (i, 0)),
            pl.BlockSpec((EB, D), lambda i: (i, 0)),
        ],
        out_shape=[
            jax.ShapeDtypeStruct((E, D), jnp.float32),
            jax.ShapeDtypeStruct((E, D), jnp.float32),
        ],
    )(edge_attr, ew0, eb0[None, :], ew1, eb1[None, :])


NB = 2000
NGRID = N // NB


def _stats_body(xin, agg, mw, mb, t, sums, sumsq):
    srow = xin[...] + agg[0] + agg[1]
    tv = jnp.dot(srow, mw[...], preferred_element_type=jnp.float32) + mb[...]
    t[...] = tv
    sums[...] = jnp.sum(tv, axis=0, keepdims=True)[None]
    sumsq[...] = jnp.sum(tv * tv, axis=0, keepdims=True)[None]


def _node_stats(xin, agg, mw, mb):
    return pl.pallas_call(
        _stats_body,
        grid=(NGRID,),
        in_specs=[
            pl.BlockSpec((NB, D), lambda i: (i, 0)),
            pl.BlockSpec((NC, NB, D), lambda i: (0, i, 0)),
            pl.BlockSpec((D, D), lambda i: (0, 0)),
            pl.BlockSpec((1, D), lambda i: (0, 0)),
        ],
        out_specs=[
            pl.BlockSpec((NB, D), lambda i: (i, 0)),
            pl.BlockSpec((1, 1, D), lambda i: (i, 0, 0)),
            pl.BlockSpec((1, 1, D), lambda i: (i, 0, 0)),
        ],
        out_shape=[
            jax.ShapeDtypeStruct((N, D), jnp.float32),
            jax.ShapeDtypeStruct((NGRID, 1, D), jnp.float32),
            jax.ShapeDtypeStruct((NGRID, 1, D), jnp.float32),
        ],
    )(xin, agg, mw, mb[None, :])


def _finish_body(t, scale, shift, onehot, h, pool):
    hv = jnp.maximum(t[...] * scale[...] + shift[...], 0.0)
    h[...] = hv
    pool[...] = jax.lax.dot_general(
        onehot[...], hv, (((0,), (0,)), ((), ())),
        preferred_element_type=jnp.float32)[None]


def _node_finish(t, scale, shift, onehot):
    return pl.pallas_call(
        _finish_body,
        grid=(NGRID,),
        in_specs=[
            pl.BlockSpec((NB, D), lambda i: (i, 0)),
            pl.BlockSpec((1, D), lambda i: (0, 0)),
            pl.BlockSpec((1, D), lambda i: (0, 0)),
            pl.BlockSpec((NB, G), lambda i: (i, 0)),
        ],
        out_specs=[
            pl.BlockSpec((NB, D), lambda i: (i, 0)),
            pl.BlockSpec((1, G, D), lambda i: (i, 0, 0)),
        ],
        out_shape=[
            jax.ShapeDtypeStruct((N, D), jnp.float32),
            jax.ShapeDtypeStruct((NGRID, G, D), jnp.float32),
        ],
    )(t, scale, shift, onehot)


def _layer(xin, e, src, dst, mw, mb, g, b, onehot):
    agg = _sc_message_pass(xin, e, src, dst)
    t, sums, sumsq = _node_stats(xin, agg, mw, mb)
    mu = jnp.sum(sums[:, 0], axis=0) / N
    var = jnp.sum(sumsq[:, 0], axis=0) / N - mu * mu
    inv = lax.rsqrt(var + 1e-5)
    scale = g * inv
    shift = b - mu * scale
    h, pool = _node_finish(t, scale[None, :], shift[None, :], onehot)
    return h, jnp.sum(pool, axis=0)


def kernel(x, edge_index, seq_batch_node_id, edge_attr,
           edge_w0, edge_b0, mlp_w0, mlp_b0, bn_g0, bn_b0,
           edge_w1, edge_b1, mlp_w1, mlp_b1, bn_g1, bn_b1):
    src = edge_index[0]
    dst = edge_index[1]
    e0, e1 = _edge_transform(edge_attr, edge_w0, edge_b0, edge_w1, edge_b1)
    onehot = (seq_batch_node_id[:, None] ==
              jnp.arange(G, dtype=seq_batch_node_id.dtype)[None, :]
              ).astype(jnp.float32)
    h0, p0 = _layer(x, e0, src, dst, mlp_w0, mlp_b0, bn_g0, bn_b0, onehot)
    h1, p1 = _layer(h0, e1, src, dst, mlp_w1, mlp_b1, bn_g1, bn_b1, onehot)
    return jnp.concatenate([p0, p1], axis=1)


# speedup vs baseline: 6.1338x; 1.8472x over previous
import functools

import jax
import jax.numpy as jnp
from jax import lax
from jax.experimental import pallas as pl
from jax.experimental.pallas import tpu as pltpu
from jax.experimental.pallas import tpu_sc as plsc

N = 10000
E = 640000
D = 128
G = 16

NC = 2
NS = 16
NW = NC * NS
EPW = E // NW
C = 80
CH = EPW // C
ZROWS = N // C
RPT = N // NS


NG = CH // 2


def _sc_body(x_hbm, e_hbm, src4_hbm, dst4_hbm, out_hbm,
             sidx, didx, rows, erows, agg,
             sem_in0, sem_in1, sem_out0, sem_out1, sem_idx):
    c = lax.axis_index("c")
    s = lax.axis_index("s")
    w = c * NS + s
    ebase = w * EPW
    sems_in = (sem_in0, sem_in1)
    sems_out = (sem_out0, sem_out1)

    def issue_idx(p):
        q = lax.rem(p, 2)
        pltpu.async_copy(src4_hbm.at[w, p], sidx.at[q], sem_idx)
        pltpu.async_copy(dst4_hbm.at[w, p], didx.at[q], sem_idx)

    def wait_idx():
        pltpu.make_async_copy(src4_hbm.at[w, 0], sidx.at[0], sem_idx).wait()
        pltpu.make_async_copy(src4_hbm.at[w, 0], didx.at[0], sem_idx).wait()

    def issue_in(i, b):
        q = lax.rem(lax.div(i, 2), 2)
        k = lax.rem(i, 2)
        pltpu.async_copy(x_hbm.at[sidx.at[q, k]], rows.at[b], sems_in[b])
        pltpu.async_copy(e_hbm.at[pl.ds(ebase + i * C, C)], erows.at[b],
                         sems_in[b])

    def wait_in(b):
        pltpu.make_async_copy(x_hbm.at[pl.ds(0, C)], rows.at[b],
                              sems_in[b]).wait()
        pltpu.make_async_copy(x_hbm.at[pl.ds(0, C)], erows.at[b],
                              sems_in[b]).wait()

    def issue_out(i, b):
        q = lax.rem(lax.div(i, 2), 2)
        k = lax.rem(i, 2)
        pltpu.async_copy(rows.at[b], agg.at[didx.at[q, k]], sems_out[b],
                         add=True)

    def wait_out(b):
        pltpu.make_async_copy(rows.at[b], agg.at[pl.ds(0, C)],
                              sems_out[b]).wait()

    def compute(b):
        rb = rows.at[b]
        eb = erows.at[b]

        @plsc.parallel_loop(0, C)
        def _(r):
            for j in range(8):
                v = rb[r, pl.ds(j * 16, 16)] + eb[r, pl.ds(j * 16, 16)]
                rb[r, pl.ds(j * 16, 16)] = jnp.maximum(v, 0.0)

    issue_idx(0)

    @plsc.parallel_loop(0, C)
    def _(r):
        zv = jnp.zeros((16,), jnp.float32)
        for j in range(8):
            rows[0, r, pl.ds(j * 16, 16)] = zv

    for jj in range(ZROWS // NS + 1):
        j = s + jj * NS
        @pl.when(j < ZROWS)
        def _():
            pltpu.sync_copy(rows.at[0], agg.at[pl.ds(j * C, C)])

    wait_idx()
    plsc.subcore_barrier()

    issue_in(0, 0)

    def _group(g, _):
        i0 = 2 * g
        wait_in(0)

        @pl.when(g > 0)
        def _():
            wait_out(1)

        @pl.when(g + 1 < NG)
        def _():
            issue_idx(g + 1)
        issue_in(i0 + 1, 1)
        compute(0)
        issue_out(i0, 0)
        wait_in(1)

        @pl.when(g + 1 < NG)
        def _():
            wait_out(0)
            wait_idx()
            issue_in(i0 + 2, 0)
        compute(1)
        issue_out(i0 + 1, 1)
        return 0
    lax.fori_loop(0, NG, _group, 0)
    wait_out(0)
    wait_out(1)
    plsc.subcore_barrier()

    for jj in range(ZROWS // NS + 1):
        j = s + jj * NS
        @pl.when(j < ZROWS)
        def _():
            pltpu.sync_copy(agg.at[pl.ds(j * C, C)],
                            out_hbm.at[c, pl.ds(j * C, C)])


_sc_message_pass = functools.partial(
    pl.kernel,
    out_type=jax.ShapeDtypeStruct((NC, N, D), jnp.float32),
    mesh=plsc.VectorSubcoreMesh(
        core_axis_name="c", subcore_axis_name="s",
        num_cores=NC, num_subcores=NS),
    scratch_types=[
        pltpu.VMEM((2, 2, C), jnp.int32),
        pltpu.VMEM((2, 2, C), jnp.int32),
        pltpu.VMEM((2, C, D), jnp.float32),
        pltpu.VMEM((2, C, D), jnp.float32),
        pltpu.VMEM_SHARED((N, D), jnp.float32),
        pltpu.SemaphoreType.DMA,
        pltpu.SemaphoreType.DMA,
        pltpu.SemaphoreType.DMA,
        pltpu.SemaphoreType.DMA,
        pltpu.SemaphoreType.DMA,
    ],
)(_sc_body)


EB = 2000


def _edge_body(ea, w0, b0, w1, b1, e0, e1):
    a = ea[...]
    e0[...] = jnp.dot(a, w0[...], preferred_element_type=jnp.float32) + b0[...]
    e1[...] = jnp.dot(a, w1[...], preferred_element_type=jnp.float32) + b1[...]


def _edge_transform(edge_attr, ew0, eb0, ew1, eb1):
    de = edge_attr.shape[1]
    grid = E // EB
    return pl.pallas_call(
        _edge_body,
        grid=(grid,),
        in_specs=[
            pl.BlockSpec((EB, de), lambda i: (i, 0)),
            pl.BlockSpec((de, D), lambda i: (0, 0)),
            pl.BlockSpec((1, D), lambda i: (0, 0)),
            pl.BlockSpec((de, D), lambda i: (0, 0)),
            pl.BlockSpec((1, D), lambda i: (0, 0)),
        ],
        out_specs=[
            pl.BlockSpec((EB, D), lambda i: (i, 0)),
            pl.BlockSpec((EB, D), lambda i: (i, 0)),
        ],
        out_shape=[
            jax.ShapeDtypeStruct((E, D), jnp.float32),
            jax.ShapeDtypeStruct((E, D), jnp.float32),
        ],
    )(edge_attr, ew0, eb0[None, :], ew1, eb1[None, :])


NB = 2000
NGRID = N // NB


def _stats_body(xin, agg, mw, mb, t, sums, sumsq):
    srow = xin[...] + agg[0] + agg[1]
    tv = jnp.dot(srow, mw[...], preferred_element_type=jnp.float32) + mb[...]
    t[...] = tv
    sums[...] = jnp.sum(tv, axis=0, keepdims=True)[None]
    sumsq[...] = jnp.sum(tv * tv, axis=0, keepdims=True)[None]


def _node_stats(xin, agg, mw, mb):
    return pl.pallas_call(
        _stats_body,
        grid=(NGRID,),
        in_specs=[
            pl.BlockSpec((NB, D), lambda i: (i, 0)),
            pl.BlockSpec((NC, NB, D), lambda i: (0, i, 0)),
            pl.BlockSpec((D, D), lambda i: (0, 0)),
            pl.BlockSpec((1, D), lambda i: (0, 0)),
        ],
        out_specs=[
            pl.BlockSpec((NB, D), lambda i: (i, 0)),
            pl.BlockSpec((1, 1, D), lambda i: (i, 0, 0)),
            pl.BlockSpec((1, 1, D), lambda i: (i, 0, 0)),
        ],
        out_shape=[
            jax.ShapeDtypeStruct((N, D), jnp.float32),
            jax.ShapeDtypeStruct((NGRID, 1, D), jnp.float32),
            jax.ShapeDtypeStruct((NGRID, 1, D), jnp.float32),
        ],
    )(xin, agg, mw, mb[None, :])


def _finish_body(t, scale, shift, onehot, h, pool):
    hv = jnp.maximum(t[...] * scale[...] + shift[...], 0.0)
    h[...] = hv
    pool[...] = jax.lax.dot_general(
        onehot[...], hv, (((0,), (0,)), ((), ())),
        preferred_element_type=jnp.float32)[None]


def _node_finish(t, scale, shift, onehot):
    return pl.pallas_call(
        _finish_body,
        grid=(NGRID,),
        in_specs=[
            pl.BlockSpec((NB, D), lambda i: (i, 0)),
            pl.BlockSpec((1, D), lambda i: (0, 0)),
            pl.BlockSpec((1, D), lambda i: (0, 0)),
            pl.BlockSpec((NB, G), lambda i: (i, 0)),
        ],
        out_specs=[
            pl.BlockSpec((NB, D), lambda i: (i, 0)),
            pl.BlockSpec((1, G, D), lambda i: (i, 0, 0)),
        ],
        out_shape=[
            jax.ShapeDtypeStruct((N, D), jnp.float32),
            jax.ShapeDtypeStruct((NGRID, G, D), jnp.float32),
        ],
    )(t, scale, shift, onehot)


def _layer(xin, e, src, dst, mw, mb, g, b, onehot):
    agg = _sc_message_pass(xin, e, src, dst)
    t, sums, sumsq = _node_stats(xin, agg, mw, mb)
    mu = jnp.sum(sums[:, 0], axis=0) / N
    var = jnp.sum(sumsq[:, 0], axis=0) / N - mu * mu
    inv = lax.rsqrt(var + 1e-5)
    scale = g * inv
    shift = b - mu * scale
    h, pool = _node_finish(t, scale[None, :], shift[None, :], onehot)
    return h, jnp.sum(pool, axis=0)


def kernel(x, edge_index, seq_batch_node_id, edge_attr,
           edge_w0, edge_b0, mlp_w0, mlp_b0, bn_g0, bn_b0,
           edge_w1, edge_b1, mlp_w1, mlp_b1, bn_g1, bn_b1):
    src = edge_index[0].reshape(NW, NG, 2, C)
    dst = edge_index[1].reshape(NW, NG, 2, C)
    e0, e1 = _edge_transform(edge_attr, edge_w0, edge_b0, edge_w1, edge_b1)
    onehot = (seq_batch_node_id[:, None] ==
              jnp.arange(G, dtype=seq_batch_node_id.dtype)[None, :]
              ).astype(jnp.float32)
    h0, p0 = _layer(x, e0, src, dst, mlp_w0, mlp_b0, bn_g0, bn_b0, onehot)
    h1, p1 = _layer(h0, e1, src, dst, mlp_w1, mlp_b1, bn_g1, bn_b1, onehot)
    return jnp.concatenate([p0, p1], axis=1)
